# baseline (speedup 1.0000x reference)
import functools

import jax
import jax.numpy as jnp
from jax import lax
from jax.experimental import pallas as pl
from jax.experimental.pallas import tpu as pltpu
from jax.experimental.pallas import tpu_sc as plsc

BATCH = 16384
CLASSES = 100
D = 128
PADD = D + 1
LANES = 16
NUM_CORES = 2
NUM_SUBCORES = 16
NW = NUM_CORES * NUM_SUBCORES
SPW = BATCH // NW
CHUNK = 128
NCHUNK = SPW // CHUNK
NBLK = CHUNK // LANES
NK = D // LANES


def _rsqrt_newton(x):
    i = lax.bitcast_convert_type(x, jnp.int32)
    i = jnp.int32(0x5F3759DF) - lax.shift_right_logical(i, 1)
    y = lax.bitcast_convert_type(i, jnp.float32)
    for _ in range(3):
        y = y * (1.5 - 0.5 * x * y * y)
    return y


_MESH = plsc.VectorSubcoreMesh(
    core_axis_name="c", subcore_axis_name="s",
    num_cores=NUM_CORES, num_subcores=NUM_SUBCORES)


@functools.partial(
    pl.kernel,
    out_type=jax.ShapeDtypeStruct((NW, LANES), jnp.float32),
    mesh=_MESH,
    scratch_types=[
        pltpu.VMEM((CLASSES * D,), jnp.float32),
        pltpu.VMEM((CLASSES * PADD + 7,), jnp.float32),
        pltpu.VMEM((SPW,), jnp.int32),
        pltpu.VMEM((CHUNK * D,), jnp.float32),
        pltpu.VMEM((CHUNK * D,), jnp.float32),
        pltpu.VMEM((CHUNK * D,), jnp.float32),
        pltpu.VMEM((CHUNK * D,), jnp.float32),
        pltpu.VMEM((LANES,), jnp.float32),
        pltpu.SemaphoreType.DMA,
        pltpu.SemaphoreType.DMA,
    ],
    compiler_params=pltpu.CompilerParams(needs_layout_passes=False),
)
def _clop_sc(zi_hbm, zj_hbm, lbl_hbm, anc_hbm, out_hbm,
             anc_raw, anc_v, lbl_v, zi0, zi1, zj0, zj1, res_v, sem0, sem1):
    wid = lax.axis_index("s") * NUM_CORES + lax.axis_index("c")
    base = wid * SPW

    pltpu.sync_copy(anc_hbm, anc_raw)
    pltpu.sync_copy(lbl_hbm.at[pl.ds(base, SPW)], lbl_v)

    lane = lax.iota(jnp.int32, LANES)

    def relayout(r, carry):
        for k in range(NK):
            v = anc_raw[pl.ds(r * D + k * LANES, LANES)]
            plsc.store_scatter(anc_v, [r * PADD + (lane + k * LANES)], v)
        return carry

    plsc.parallel_loop(0, CLASSES, unroll=2, carry=jnp.int32(0))(relayout)

    zib = (zi0, zi1)
    zjb = (zj0, zj1)
    sems = (sem0, sem1)

    def start(g):
        b = g % 2
        off = (base + g * CHUNK) * D
        hi = pltpu.async_copy(zi_hbm.at[pl.ds(off, CHUNK * D)], zib[b], sems[b])
        hj = pltpu.async_copy(zj_hbm.at[pl.ds(off, CHUNK * D)], zjb[b], sems[b])
        return (hi, hj)

    pend = [None, None]
    pend[0] = start(0)

    zero = jnp.zeros((LANES,), jnp.float32)
    acc = zero
    cnt = zero

    for g in range(NCHUNK):
        b = g % 2
        if g + 1 < NCHUNK:
            pend[(g + 1) % 2] = start(g + 1)
        for h in pend[b]:
            h.wait()
        for t in range(NBLK):
            lbl = lbl_v[pl.ds(g * CHUNK + t * LANES, LANES)]
            valid = lbl >= 0
            ancbase = jnp.maximum(lbl, 0) * PADD
            rowbase = (lane + t * LANES) * D

            def dstep_mk(wrap):
                def dstep(d, carry, _rowbase=rowbase, _ancbase=ancbase, _b=b):
                    ssi, ssj, di, dj = carry
                    rot = d + lane
                    if wrap:
                        rot = rot & (D - 1)
                    zidx = _rowbase + rot
                    vi = plsc.load_gather(zib[_b], [zidx])
                    vj = plsc.load_gather(zjb[_b], [zidx])
                    av = plsc.load_gather(anc_v, [_ancbase + rot])
                    return (ssi + vi * vi, ssj + vj * vj,
                            di + vi * av, dj + vj * av)
                return dstep

            carry0 = plsc.parallel_loop(
                0, D - LANES, unroll=8,
                carry=(zero, zero, zero, zero))(dstep_mk(False))
            ssi, ssj, di, dj = plsc.parallel_loop(
                D - LANES, D, unroll=8, carry=carry0)(dstep_mk(True))

            ri = jnp.minimum(_rsqrt_newton(ssi), 1e12)
            rj = jnp.minimum(_rsqrt_newton(ssj), 1e12)
            per = 1.0 - 0.5 * (di * ri + dj * rj)
            acc = acc + jnp.where(valid, per, 0.0)
            cnt = cnt + jnp.where(valid, 1.0, 0.0)

    tsum = jnp.sum(acc)
    tcnt = jnp.sum(cnt)
    res_v[...] = jnp.where(lane == 0, tsum,
                           jnp.where(lane == 1, tcnt, 0.0))
    pltpu.sync_copy(res_v, out_hbm.at[wid])


@jax.jit
def kernel(z_i, z_j, labels, anchors):
    parts = _clop_sc(z_i.reshape(-1), z_j.reshape(-1),
                     labels, anchors.reshape(-1))
    total = jnp.sum(parts[:, 0])
    count = jnp.sum(parts[:, 1])
    return total / jnp.maximum(count, 1.0)


# speedup vs baseline: 1.2122x; 1.2122x over previous
import functools

import jax
import jax.numpy as jnp
from jax import lax
from jax.experimental import pallas as pl
from jax.experimental.pallas import tpu as pltpu
from jax.experimental.pallas import tpu_sc as plsc

BATCH = 16384
CLASSES = 100
D = 128
PADD = D + 1
LANES = 16
NUM_CORES = 2
NUM_SUBCORES = 16
NW = NUM_CORES * NUM_SUBCORES
SPW = BATCH // NW
CHUNK = 64
NCHUNK = SPW // CHUNK
NBLK = CHUNK // LANES
NK = D // LANES


def _rsqrt_newton(x):
    i = lax.bitcast_convert_type(x, jnp.int32)
    i = jnp.int32(0x5F3759DF) - lax.shift_right_logical(i, 1)
    y = lax.bitcast_convert_type(i, jnp.float32)
    for _ in range(3):
        y = y * (1.5 - 0.5 * x * y * y)
    return y


_MESH = plsc.VectorSubcoreMesh(
    core_axis_name="c", subcore_axis_name="s",
    num_cores=NUM_CORES, num_subcores=NUM_SUBCORES)


@functools.partial(
    pl.kernel,
    out_type=jax.ShapeDtypeStruct((NW, LANES), jnp.float32),
    mesh=_MESH,
    scratch_types=[
        pltpu.VMEM((CLASSES * D,), jnp.float32),
        pltpu.VMEM((CLASSES * PADD + 7,), jnp.float32),
        pltpu.VMEM((SPW,), jnp.int32),
        pltpu.VMEM((CHUNK * D,), jnp.float32),
        pltpu.VMEM((CHUNK * D,), jnp.float32),
        pltpu.VMEM((CHUNK * D,), jnp.float32),
        pltpu.VMEM((CHUNK * D,), jnp.float32),
        pltpu.VMEM((LANES,), jnp.float32),
        pltpu.SemaphoreType.DMA,
        pltpu.SemaphoreType.DMA,
    ],
    compiler_params=pltpu.CompilerParams(needs_layout_passes=False),
)
def _clop_sc(zi_hbm, zj_hbm, lbl_hbm, anc_hbm, out_hbm,
             anc_raw, anc_v, lbl_v, zi0, zi1, zj0, zj1, res_v, sem0, sem1):
    wid = lax.axis_index("s") * NUM_CORES + lax.axis_index("c")
    base = wid * SPW

    pltpu.sync_copy(anc_hbm, anc_raw)
    pltpu.sync_copy(lbl_hbm.at[pl.ds(base, SPW)], lbl_v)

    lane = lax.iota(jnp.int32, LANES)

    def relayout(r, carry):
        for k in range(NK):
            v = anc_raw[pl.ds(r * D + k * LANES, LANES)]
            plsc.store_scatter(anc_v, [r * PADD + (lane + k * LANES)], v)
        return carry

    plsc.parallel_loop(0, CLASSES, unroll=2, carry=jnp.int32(0))(relayout)

    zib = (zi0, zi1)
    zjb = (zj0, zj1)
    sems = (sem0, sem1)

    def start(g):
        b = g % 2
        off = (base + g * CHUNK) * D
        hi = pltpu.async_copy(zi_hbm.at[pl.ds(off, CHUNK * D)], zib[b], sems[b])
        hj = pltpu.async_copy(zj_hbm.at[pl.ds(off, CHUNK * D)], zjb[b], sems[b])
        return (hi, hj)

    pend = [None, None]
    pend[0] = start(0)

    zero = jnp.zeros((LANES,), jnp.float32)
    acc = zero
    cnt = zero

    for g in range(NCHUNK):
        b = g % 2
        if g + 1 < NCHUNK:
            pend[(g + 1) % 2] = start(g + 1)
        for h in pend[b]:
            h.wait()

        def block(t, carry, _g=g, _b=b):
            acc, cnt = carry
            lbl = lbl_v[pl.ds(_g * CHUNK + t * LANES, LANES)]
            valid = lbl >= 0
            ancbase = jnp.maximum(lbl, 0) * PADD
            rowbase = lane * D + t * (LANES * D)

            def dstep(d, dcarry, _rowbase=rowbase, _ancbase=ancbase):
                ssi, ssj, di, dj = dcarry
                rot = (d + lane) & (D - 1)
                zidx = _rowbase + rot
                vi = plsc.load_gather(zib[_b], [zidx])
                vj = plsc.load_gather(zjb[_b], [zidx])
                av = plsc.load_gather(anc_v, [_ancbase + rot])
                return (ssi + vi * vi, ssj + vj * vj,
                        di + vi * av, dj + vj * av)

            ssi, ssj, di, dj = plsc.parallel_loop(
                0, D, unroll=4, carry=(zero, zero, zero, zero))(dstep)

            ri = jnp.minimum(_rsqrt_newton(ssi), 1e12)
            rj = jnp.minimum(_rsqrt_newton(ssj), 1e12)
            per = 1.0 - 0.5 * (di * ri + dj * rj)
            acc = acc + jnp.where(valid, per, 0.0)
            cnt = cnt + jnp.where(valid, 1.0, 0.0)
            return (acc, cnt)

        acc, cnt = lax.fori_loop(0, NBLK, block, (acc, cnt))

    tsum = jnp.sum(acc)
    tcnt = jnp.sum(cnt)
    res_v[...] = jnp.where(lane == 0, tsum,
                           jnp.where(lane == 1, tcnt, 0.0))
    pltpu.sync_copy(res_v, out_hbm.at[wid])


@jax.jit
def kernel(z_i, z_j, labels, anchors):
    parts = _clop_sc(z_i.reshape(-1), z_j.reshape(-1),
                     labels, anchors.reshape(-1))
    total = jnp.sum(parts[:, 0])
    count = jnp.sum(parts[:, 1])
    return total / jnp.maximum(count, 1.0)


# speedup vs baseline: 1.2137x; 1.0012x over previous
import functools

import jax
import jax.numpy as jnp
from jax import lax
from jax.experimental import pallas as pl
from jax.experimental.pallas import tpu as pltpu
from jax.experimental.pallas import tpu_sc as plsc

BATCH = 16384
CLASSES = 100
D = 128
PADD = D + 1
LANES = 16
NUM_CORES = 2
NUM_SUBCORES = 16
NW = NUM_CORES * NUM_SUBCORES
SPW = BATCH // NW
CHUNK = 64
NCHUNK = SPW // CHUNK
NBLK = CHUNK // LANES
NK = D // LANES


def _rsqrt_newton(x):
    i = lax.bitcast_convert_type(x, jnp.int32)
    i = jnp.int32(0x5F3759DF) - lax.shift_right_logical(i, 1)
    y = lax.bitcast_convert_type(i, jnp.float32)
    for _ in range(3):
        y = y * (1.5 - 0.5 * x * y * y)
    return y


_MESH = plsc.VectorSubcoreMesh(
    core_axis_name="c", subcore_axis_name="s",
    num_cores=NUM_CORES, num_subcores=NUM_SUBCORES)


@functools.partial(
    pl.kernel,
    out_type=jax.ShapeDtypeStruct((NW, LANES), jnp.float32),
    mesh=_MESH,
    scratch_types=[
        pltpu.VMEM((CLASSES * D,), jnp.float32),
        pltpu.VMEM((CLASSES * PADD + 7,), jnp.float32),
        pltpu.VMEM((SPW,), jnp.int32),
        pltpu.VMEM((CHUNK, D), jnp.float32),
        pltpu.VMEM((CHUNK, D), jnp.float32),
        pltpu.VMEM((CHUNK, D), jnp.float32),
        pltpu.VMEM((CHUNK, D), jnp.float32),
        pltpu.VMEM((LANES,), jnp.float32),
        pltpu.SemaphoreType.DMA,
        pltpu.SemaphoreType.DMA,
    ],
    compiler_params=pltpu.CompilerParams(needs_layout_passes=False),
)
def _clop_sc(zi_hbm, zj_hbm, lbl_hbm, anc_hbm, out_hbm,
             anc_raw, anc_v, lbl_v, zi0, zi1, zj0, zj1, res_v, sem0, sem1):
    wid = lax.axis_index("s") * NUM_CORES + lax.axis_index("c")
    base = wid * SPW

    pltpu.sync_copy(anc_hbm, anc_raw)
    pltpu.sync_copy(lbl_hbm.at[pl.ds(base, SPW)], lbl_v)

    lane = lax.iota(jnp.int32, LANES)

    def relayout(r, carry):
        for k in range(NK):
            v = anc_raw[pl.ds(r * D + k * LANES, LANES)]
            plsc.store_scatter(anc_v, [r * PADD + (lane + k * LANES)], v)
        return carry

    plsc.parallel_loop(0, CLASSES, unroll=2, carry=jnp.int32(0))(relayout)

    zib = (zi0, zi1)
    zjb = (zj0, zj1)
    sems = (sem0, sem1)

    def start(g):
        b = g % 2
        off = base + g * CHUNK
        hi = pltpu.async_copy(
            zi_hbm.at[pl.ds(off, CHUNK), :], zib[b], sems[b])
        hj = pltpu.async_copy(
            zj_hbm.at[pl.ds(off, CHUNK), :], zjb[b], sems[b])
        return (hi, hj)

    pend = [None, None]
    pend[0] = start(0)

    zero = jnp.zeros((LANES,), jnp.float32)
    acc = zero
    cnt = zero

    for g in range(NCHUNK):
        b = g % 2
        if g + 1 < NCHUNK:
            pend[(g + 1) % 2] = start(g + 1)
        for h in pend[b]:
            h.wait()

        def block(t, carry, _g=g, _b=b):
            acc, cnt = carry
            lbl = lbl_v[pl.ds(_g * CHUNK + t * LANES, LANES)]
            valid = lbl >= 0
            ancbase = jnp.maximum(lbl, 0) * PADD
            rows = lane + t * LANES

            def dstep(d, dcarry, _rows=rows, _ancbase=ancbase):
                ssi, ssj, di, dj = dcarry
                rot = (d + lane) & (D - 1)
                vi = plsc.load_gather(zib[_b], [_rows, rot])
                vj = plsc.load_gather(zjb[_b], [_rows, rot])
                av = plsc.load_gather(anc_v, [_ancbase + rot])
                return (ssi + vi * vi, ssj + vj * vj,
                        di + vi * av, dj + vj * av)

            ssi, ssj, di, dj = plsc.parallel_loop(
                0, D, unroll=4, carry=(zero, zero, zero, zero))(dstep)

            ri = jnp.minimum(_rsqrt_newton(ssi), 1e12)
            rj = jnp.minimum(_rsqrt_newton(ssj), 1e12)
            per = 1.0 - 0.5 * (di * ri + dj * rj)
            acc = acc + jnp.where(valid, per, 0.0)
            cnt = cnt + jnp.where(valid, 1.0, 0.0)
            return (acc, cnt)

        acc, cnt = lax.fori_loop(0, NBLK, block, (acc, cnt))

    tsum = jnp.sum(acc)
    tcnt = jnp.sum(cnt)
    res_v[...] = jnp.where(lane == 0, tsum,
                           jnp.where(lane == 1, tcnt, 0.0))
    pltpu.sync_copy(res_v, out_hbm.at[wid])


@jax.jit
def kernel(z_i, z_j, labels, anchors):
    parts = _clop_sc(z_i, z_j, labels, anchors.reshape(-1))
    total = jnp.sum(parts[:, 0])
    count = jnp.sum(parts[:, 1])
    return total / jnp.maximum(count, 1.0)
